# baseline (speedup 1.0000x reference)
import functools

import jax
import jax.numpy as jnp
from jax import lax
from jax.experimental import pallas as pl
from jax.experimental.pallas import tpu as pltpu
from jax.experimental.pallas import tpu_sc as plsc

_NF = 16
_H = 1489
_W = 2048
_B = 128
_HLEN = 256
_WLEN = 256
_HOFF = 500
_WOFF = 700

_RBLK = 496
_CBLK = 640


def _crop_body(bg_ref, out_ref):
    out_ref[0] = bg_ref[
        0, 0, pl.ds(_HOFF - _RBLK, _HLEN), pl.ds(_WOFF - _CBLK, _WLEN)
    ]


def _crop(background):
    return pl.pallas_call(
        _crop_body,
        grid=(_NF,),
        in_specs=[pl.BlockSpec((1, 1, _RBLK, _CBLK), lambda i: (i, 0, 1, 1))],
        out_specs=pl.BlockSpec((1, _HLEN, _WLEN), lambda i: (i, 0, 0)),
        out_shape=jax.ShapeDtypeStruct((_NF, _HLEN, _WLEN), jnp.float32),
    )(background)


_GPT = _HLEN // 8
_HGRP = _GPT // 2


def _sc_gather(table3, rcf_indices):
    nc, ns = 2, 16
    nw = nc * ns
    bpw = _B // nw

    mesh = plsc.VectorSubcoreMesh(core_axis_name="c", subcore_axis_name="s")

    @functools.partial(
        pl.kernel,
        out_type=jax.ShapeDtypeStruct((_B * _GPT, 8, _WLEN), jnp.float32),
        mesh=mesh,
        compiler_params=pltpu.CompilerParams(needs_layout_passes=False),
        scratch_types=[
            pltpu.VMEM((_B,), jnp.int32),
        ],
    )
    def k(tab, idx, out, idx_v):
        wid = lax.axis_index("s") * nc + lax.axis_index("c")
        pltpu.sync_copy(idx, idx_v)
        for jj in range(bpw):
            b = wid * bpw + jj
            fvec = plsc.load_gather(idx_v, [jnp.full((16,), b, jnp.int32)])
            f = fvec[0]
            for h in range(2):
                pltpu.sync_copy(
                    tab.at[pl.ds(f * _GPT + h * _HGRP, _HGRP)],
                    out.at[pl.ds(b * _GPT + h * _HGRP, _HGRP)],
                )

    return k(table3, rcf_indices)


def kernel(background, rcf_indices):
    table3 = _crop(background).reshape(_NF * _GPT, 8, _WLEN)
    out3 = _sc_gather(table3, rcf_indices)
    return out3.reshape(_B, 1, _HLEN, _WLEN)


# speedup vs baseline: 6.4536x; 6.4536x over previous
import functools

import jax
import jax.numpy as jnp
from jax import lax
from jax.experimental import pallas as pl
from jax.experimental.pallas import tpu as pltpu
from jax.experimental.pallas import tpu_sc as plsc

_NF = 16
_H = 1489
_W = 2048
_B = 128
_HLEN = 256
_WLEN = 256
_HOFF = 500
_WOFF = 700

_RBLK = 496
_CBLK = 640


def _crop_body(bg_ref, out_ref):
    out_ref[0] = bg_ref[
        0, 0, pl.ds(_HOFF - _RBLK, _HLEN), pl.ds(_WOFF - _CBLK, _WLEN)
    ]


def _crop(background):
    return pl.pallas_call(
        _crop_body,
        grid=(_NF,),
        in_specs=[pl.BlockSpec((1, 1, _RBLK, _CBLK), lambda i: (i, 0, 1, 1))],
        out_specs=pl.BlockSpec((1, _HLEN, _WLEN), lambda i: (i, 0, 0)),
        out_shape=jax.ShapeDtypeStruct((_NF, _HLEN, _WLEN), jnp.float32),
    )(background)


_GPT = _HLEN // 8
_HGRP = _GPT // 2


def _sc_gather(table3, rcf_indices):
    nc, ns = 2, 16
    nw = nc * ns
    bpw = _B // nw

    mesh = plsc.VectorSubcoreMesh(core_axis_name="c", subcore_axis_name="s")

    @functools.partial(
        pl.kernel,
        out_type=jax.ShapeDtypeStruct((_B * _GPT, 8, _WLEN), jnp.float32),
        mesh=mesh,
        compiler_params=pltpu.CompilerParams(needs_layout_passes=False),
        scratch_types=[
            pltpu.VMEM((_B,), jnp.int32),
            pltpu.VMEM((3, _HGRP, 8, _WLEN), jnp.float32),
            pltpu.SemaphoreType.DMA,
            pltpu.SemaphoreType.DMA,
            pltpu.SemaphoreType.DMA,
            pltpu.SemaphoreType.DMA,
            pltpu.SemaphoreType.DMA,
            pltpu.SemaphoreType.DMA,
        ],
    )
    def k(tab, idx, out, idx_v, bufs, si0, si1, si2, so0, so1, so2):
        wid = lax.axis_index("s") * nc + lax.axis_index("c")
        pltpu.sync_copy(idx, idx_v)
        in_sems = [si0, si1, si2]
        out_sems = [so0, so1, so2]
        nchunks = bpw * 2
        fvals = []
        for jj in range(bpw):
            b = wid * bpw + jj
            fvec = plsc.load_gather(idx_v, [jnp.full((16,), b, jnp.int32)])
            fvals.append(fvec[0])
        pending_in = [None] * 3
        pending_out = [None] * 3
        for j in range(nchunks + 1):
            if j < nchunks:
                jj, h = j // 2, j % 2
                s = j % 3
                if pending_out[s] is not None:
                    pending_out[s].wait()
                pending_in[s] = pltpu.async_copy(
                    tab.at[pl.ds(fvals[jj] * _GPT + h * _HGRP, _HGRP)],
                    bufs.at[s],
                    in_sems[s],
                )
            if j >= 1:
                jp, s2 = j - 1, (j - 1) % 3
                jjp, hp = jp // 2, jp % 2
                bp = wid * bpw + jjp
                pending_in[s2].wait()
                pending_out[s2] = pltpu.async_copy(
                    bufs.at[s2],
                    out.at[pl.ds(bp * _GPT + hp * _HGRP, _HGRP)],
                    out_sems[s2],
                )
        for s in range(3):
            if pending_out[s] is not None:
                pending_out[s].wait()

    return k(table3, rcf_indices)


def kernel(background, rcf_indices):
    table3 = _crop(background).reshape(_NF * _GPT, 8, _WLEN)
    out3 = _sc_gather(table3, rcf_indices)
    return out3.reshape(_B, 1, _HLEN, _WLEN)


# speedup vs baseline: 6.4767x; 1.0036x over previous
import functools

import jax
import jax.numpy as jnp
from jax import lax
from jax.experimental import pallas as pl
from jax.experimental.pallas import tpu as pltpu
from jax.experimental.pallas import tpu_sc as plsc

_NF = 16
_H = 1489
_W = 2048
_B = 128
_HLEN = 256
_WLEN = 256
_HOFF = 500
_WOFF = 700

_RBLK = 496
_CBLK = 640


def _crop_body(bg_ref, out_ref):
    out_ref[0] = bg_ref[
        0, 0, pl.ds(_HOFF - _RBLK, _HLEN), pl.ds(_WOFF - _CBLK, _WLEN)
    ]


def _crop(background):
    return pl.pallas_call(
        _crop_body,
        grid=(_NF,),
        in_specs=[pl.BlockSpec((1, 1, _RBLK, _CBLK), lambda i: (i, 0, 1, 1))],
        out_specs=pl.BlockSpec((1, _HLEN, _WLEN), lambda i: (i, 0, 0)),
        out_shape=jax.ShapeDtypeStruct((_NF, _HLEN, _WLEN), jnp.float32),
    )(background)


_CH = 128


def _sc_gather(table, rcf_indices):
    nc, ns = 2, 16
    nw = nc * ns
    bpw = _B // nw

    mesh = plsc.VectorSubcoreMesh(core_axis_name="c", subcore_axis_name="s")

    @functools.partial(
        pl.kernel,
        out_type=jax.ShapeDtypeStruct((_B, 1, _HLEN, _WLEN), jnp.float32),
        mesh=mesh,
        compiler_params=pltpu.CompilerParams(needs_layout_passes=False),
        scratch_types=[
            pltpu.VMEM((_B,), jnp.int32),
            pltpu.VMEM((3, _CH, _WLEN), jnp.float32),
            pltpu.SemaphoreType.DMA,
            pltpu.SemaphoreType.DMA,
            pltpu.SemaphoreType.DMA,
            pltpu.SemaphoreType.DMA,
            pltpu.SemaphoreType.DMA,
            pltpu.SemaphoreType.DMA,
        ],
    )
    def k(tab, idx, out, idx_v, bufs, si0, si1, si2, so0, so1, so2):
        wid = lax.axis_index("s") * nc + lax.axis_index("c")
        pltpu.sync_copy(idx, idx_v)
        in_sems = [si0, si1, si2]
        out_sems = [so0, so1, so2]
        nchunks = bpw * 2
        fvals = []
        for jj in range(bpw):
            b = wid * bpw + jj
            fvec = plsc.load_gather(idx_v, [jnp.full((16,), b, jnp.int32)])
            fvals.append(fvec[0])
        pending_in = [None] * 3
        pending_out = [None] * 3
        for j in range(nchunks + 1):
            if j < nchunks:
                jj, h = j // 2, j % 2
                s = j % 3
                if pending_out[s] is not None:
                    pending_out[s].wait()
                pending_in[s] = pltpu.async_copy(
                    tab.at[fvals[jj], pl.ds(h * _CH, _CH), :],
                    bufs.at[s],
                    in_sems[s],
                )
            if j >= 1:
                jp, s2 = j - 1, (j - 1) % 3
                jjp, hp = jp // 2, jp % 2
                bp = wid * bpw + jjp
                pending_in[s2].wait()
                pending_out[s2] = pltpu.async_copy(
                    bufs.at[s2],
                    out.at[bp, 0, pl.ds(hp * _CH, _CH), :],
                    out_sems[s2],
                )
        for s in range(3):
            if pending_out[s] is not None:
                pending_out[s].wait()

    return k(table, rcf_indices)


def kernel(background, rcf_indices):
    return _sc_gather(_crop(background), rcf_indices)


# speedup vs baseline: 18.2897x; 2.8239x over previous
import functools

import jax
import jax.numpy as jnp
from jax import lax
from jax.experimental import pallas as pl
from jax.experimental.pallas import tpu as pltpu
from jax.experimental.pallas import tpu_sc as plsc

_NF = 16
_H = 1489
_W = 2048
_B = 128
_HLEN = 256
_WLEN = 256
_HOFF = 500
_WOFF = 700

_RA = _HOFF - _HOFF % 8
_CA = _WOFF - _WOFF % 128
_RSPAN = 264
_CSPAN = 384


def _crop_body(slab_ref, out_ref):
    out_ref[0] = slab_ref[
        0, 0, pl.ds(_HOFF - _RA, _HLEN), pl.ds(_WOFF - _CA, _WLEN)
    ]


def _crop(background):
    slab = lax.slice(
        background,
        (0, 0, _RA, _CA),
        (_NF, 1, _RA + _RSPAN, _CA + _CSPAN),
    )
    return pl.pallas_call(
        _crop_body,
        grid=(_NF,),
        in_specs=[pl.BlockSpec((1, 1, _RSPAN, _CSPAN), lambda i: (i, 0, 0, 0))],
        out_specs=pl.BlockSpec((1, _HLEN, _WLEN), lambda i: (i, 0, 0)),
        out_shape=jax.ShapeDtypeStruct((_NF, _HLEN, _WLEN), jnp.float32),
    )(slab)


_CH = 128


def _sc_gather(table, rcf_indices):
    nc, ns = 2, 16
    nw = nc * ns
    bpw = _B // nw

    mesh = plsc.VectorSubcoreMesh(core_axis_name="c", subcore_axis_name="s")

    @functools.partial(
        pl.kernel,
        out_type=jax.ShapeDtypeStruct((_B, 1, _HLEN, _WLEN), jnp.float32),
        mesh=mesh,
        compiler_params=pltpu.CompilerParams(needs_layout_passes=False),
        scratch_types=[
            pltpu.VMEM((_B,), jnp.int32),
            pltpu.VMEM((3, _CH, _WLEN), jnp.float32),
            pltpu.SemaphoreType.DMA,
            pltpu.SemaphoreType.DMA,
            pltpu.SemaphoreType.DMA,
            pltpu.SemaphoreType.DMA,
            pltpu.SemaphoreType.DMA,
            pltpu.SemaphoreType.DMA,
        ],
    )
    def k(tab, idx, out, idx_v, bufs, si0, si1, si2, so0, so1, so2):
        wid = lax.axis_index("s") * nc + lax.axis_index("c")
        pltpu.sync_copy(idx, idx_v)
        in_sems = [si0, si1, si2]
        out_sems = [so0, so1, so2]
        nchunks = bpw * 2
        fvals = []
        for jj in range(bpw):
            b = wid * bpw + jj
            fvec = plsc.load_gather(idx_v, [jnp.full((16,), b, jnp.int32)])
            fvals.append(fvec[0])
        pending_in = [None] * 3
        pending_out = [None] * 3
        for j in range(nchunks + 1):
            if j < nchunks:
                jj, h = j // 2, j % 2
                s = j % 3
                if pending_out[s] is not None:
                    pending_out[s].wait()
                pending_in[s] = pltpu.async_copy(
                    tab.at[fvals[jj], pl.ds(h * _CH, _CH), :],
                    bufs.at[s],
                    in_sems[s],
                )
            if j >= 1:
                jp, s2 = j - 1, (j - 1) % 3
                jjp, hp = jp // 2, jp % 2
                bp = wid * bpw + jjp
                pending_in[s2].wait()
                pending_out[s2] = pltpu.async_copy(
                    bufs.at[s2],
                    out.at[bp, 0, pl.ds(hp * _CH, _CH), :],
                    out_sems[s2],
                )
        for s in range(3):
            if pending_out[s] is not None:
                pending_out[s].wait()

    return k(table, rcf_indices)


def kernel(background, rcf_indices):
    return _sc_gather(_crop(background), rcf_indices)
